# Initial kernel scaffold; baseline (speedup 1.0000x reference)
import jax
import jax.numpy as jnp
from jax import lax
from jax.experimental import pallas as pl
from jax.experimental.pallas import tpu as pltpu
from jax.experimental.pallas import tpu_sc as plsc

B = 16384
DIM = 64
NBND = 1000
OUT_D = 2 * DIM + 1
NC, NS, L = 2, 16, 16
NW = NC * NS
BPW = B // NW
CHUNK = 128
NCH = BPW // CHUNK
NVEC = BPW // L


def _body(uid_hbm, ts_hbm, utab_hbm, ttab_hbm, bnd_hbm, mean_hbm, std_hbm,
          out_hbm, idx_v, ts_v, bnd_v, bkt_v, nrm_v, ue_v, te_v, mstd_v,
          gsem):
    wid = lax.axis_index("subcore") * NC + lax.axis_index("core")
    base = wid * BPW

    pltpu.sync_copy(uid_hbm.at[pl.ds(base, BPW)], idx_v)
    gathers = []
    for c in range(NCH):
        gathers.append(pltpu.async_copy(
            utab_hbm.at[idx_v.at[pl.ds(c * CHUNK, CHUNK)]],
            ue_v.at[pl.ds(c * CHUNK, CHUNK)], gsem))

    pltpu.sync_copy(ts_hbm.at[pl.ds(base, BPW)], ts_v)
    pltpu.sync_copy(bnd_hbm, bnd_v)
    pltpu.sync_copy(mean_hbm, mstd_v.at[0])
    pltpu.sync_copy(std_hbm, mstd_v.at[1])
    mean = mstd_v[0, :]
    std = mstd_v[1, :]

    @pl.loop(0, NVEC)
    def _(m):
        off = pl.multiple_of(m * L, L)
        ts = ts_v[pl.ds(off, L)]
        lo = jnp.zeros((L,), jnp.int32)
        hi = jnp.full((L,), NBND, jnp.int32)
        for _ in range(10):
            mid = (lo + hi) // 2
            bv = plsc.load_gather(bnd_v, [jnp.minimum(mid, NBND - 1)])
            act = lo < hi
            go = act & (bv <= ts)
            lo = jnp.where(go, mid + 1, lo)
            hi = jnp.where(act & jnp.logical_not(go), mid, hi)
        bkt_v[pl.ds(off, L)] = lo
        rid = off + lax.iota(jnp.int32, L)
        plsc.store_scatter(nrm_v, [rid, jnp.zeros((L,), jnp.int32)],
                           (ts - mean) / std)

    for c in range(NCH):
        gathers.append(pltpu.async_copy(
            ttab_hbm.at[bkt_v.at[pl.ds(c * CHUNK, CHUNK)]],
            te_v.at[pl.ds(c * CHUNK, CHUNK)], gsem))
    for g in gathers:
        g.wait()

    pltpu.sync_copy(ue_v, out_hbm.at[pl.ds(base, BPW), pl.ds(0, DIM)])
    pltpu.sync_copy(te_v, out_hbm.at[pl.ds(base, BPW), pl.ds(DIM, DIM)])
    pltpu.sync_copy(nrm_v, out_hbm.at[pl.ds(base, BPW), pl.ds(2 * DIM, 1)])


@jax.jit
def _run(user_id, time_stamp, user_table, ts_table, boundaries, mean16,
         std16):
    mesh = plsc.VectorSubcoreMesh(core_axis_name="core",
                                  subcore_axis_name="subcore")
    f = pl.kernel(
        _body,
        out_type=jax.ShapeDtypeStruct((B, OUT_D), jnp.float32),
        mesh=mesh,
        scratch_types=[
            pltpu.VMEM((BPW,), jnp.int32),
            pltpu.VMEM((BPW,), jnp.float32),
            pltpu.VMEM((NBND,), jnp.float32),
            pltpu.VMEM((BPW,), jnp.int32),
            pltpu.VMEM((BPW, 1), jnp.float32),
            pltpu.VMEM((BPW, DIM), jnp.float32),
            pltpu.VMEM((BPW, DIM), jnp.float32),
            pltpu.VMEM((2, L), jnp.float32),
            pltpu.SemaphoreType.DMA,
        ],
    )
    return f(user_id, time_stamp, user_table, ts_table, boundaries, mean16,
             std16)


def kernel(user_id, time_stamp, user_table, ts_table, boundaries, ts_mean,
           ts_std):
    mean16 = jnp.full((L,), ts_mean, dtype=jnp.float32)
    std16 = jnp.full((L,), ts_std, dtype=jnp.float32)
    return _run(user_id.astype(jnp.int32), time_stamp, user_table, ts_table,
                boundaries, mean16, std16)


# speedup vs baseline: 1.6243x; 1.6243x over previous
import jax
import jax.numpy as jnp
from jax import lax
from jax.experimental import pallas as pl
from jax.experimental.pallas import tpu as pltpu
from jax.experimental.pallas import tpu_sc as plsc

B = 16384
DIM = 64
NBND = 1000
OUT_D = 2 * DIM + 1
NC, NS, L = 2, 16, 16
NW = NC * NS
BPW = B // NW
CHUNK = 128
NCH = BPW // CHUNK
NVEC = BPW // L
HALF = BPW // 2


def _body(uid_hbm, ts_hbm, utab_hbm, ttab_hbm, bnd_hbm, mean_hbm, std_hbm,
          out_hbm, idx_v, ts_v, bnd_v, bkt_v, nrm_v, ue_v, te_v, mstd_v,
          gsem):
    wid = lax.axis_index("subcore") * NC + lax.axis_index("core")
    base = wid * BPW

    pltpu.sync_copy(uid_hbm.at[pl.ds(base, BPW)], idx_v)
    gathers = []
    for c in range(NCH):
        gathers.append(pltpu.async_copy(
            utab_hbm.at[idx_v.at[pl.ds(c * CHUNK, CHUNK)]],
            ue_v.at[pl.ds(c * CHUNK, CHUNK)], gsem))

    pltpu.sync_copy(ts_hbm.at[pl.ds(base, BPW)], ts_v)
    pltpu.sync_copy(bnd_hbm, bnd_v)
    pltpu.sync_copy(mean_hbm, mstd_v.at[0])
    pltpu.sync_copy(std_hbm, mstd_v.at[1])
    mean = mstd_v[0, :]
    std = mstd_v[1, :]

    @pl.loop(0, NVEC)
    def _(m):
        off = pl.multiple_of(m * L, L)
        ts = ts_v[pl.ds(off, L)]
        lo = jnp.zeros((L,), jnp.int32)
        hi = jnp.full((L,), NBND, jnp.int32)
        for _ in range(10):
            mid = (lo + hi) // 2
            bv = plsc.load_gather(bnd_v, [jnp.minimum(mid, NBND - 1)])
            act = lo < hi
            go = act & (bv <= ts)
            lo = jnp.where(go, mid + 1, lo)
            hi = jnp.where(act & jnp.logical_not(go), mid, hi)
        bkt_v[pl.ds(off, L)] = lo
        rid = off + lax.iota(jnp.int32, L)
        plsc.store_scatter(nrm_v, [rid, jnp.zeros((L,), jnp.int32)],
                           (ts - mean) / std)

    for c in range(NCH):
        gathers.append(pltpu.async_copy(
            ttab_hbm.at[bkt_v.at[pl.ds(c * CHUNK, CHUNK)]],
            te_v.at[pl.ds(c * CHUNK, CHUNK)], gsem))
    for g in gathers:
        g.wait()

    pltpu.sync_copy(ue_v, out_hbm.at[pl.ds(base, BPW), pl.ds(0, DIM)])
    pltpu.sync_copy(te_v, out_hbm.at[pl.ds(base, BPW), pl.ds(DIM, DIM)])
    pltpu.sync_copy(nrm_v, out_hbm.at[pl.ds(base, BPW), pl.ds(2 * DIM, 1)])


@jax.jit
def _run(user_id, time_stamp, user_table, ts_table, boundaries, mean16,
         std16):
    mesh = plsc.VectorSubcoreMesh(core_axis_name="core",
                                  subcore_axis_name="subcore")
    f = pl.kernel(
        _body,
        out_type=jax.ShapeDtypeStruct((B, OUT_D), jnp.float32),
        mesh=mesh,
        scratch_types=[
            pltpu.VMEM((BPW,), jnp.int32),
            pltpu.VMEM((BPW,), jnp.float32),
            pltpu.VMEM((NBND,), jnp.float32),
            pltpu.VMEM((BPW,), jnp.int32),
            pltpu.VMEM((BPW, 1), jnp.float32),
            pltpu.VMEM((BPW, DIM), jnp.float32),
            pltpu.VMEM((BPW, DIM), jnp.float32),
            pltpu.VMEM((2, L), jnp.float32),
            pltpu.SemaphoreType.DMA,
        ],
        compiler_params=pltpu.CompilerParams(use_tc_tiling_on_sc=False,
                                             needs_layout_passes=False),
    )
    return f(user_id, time_stamp, user_table, ts_table, boundaries, mean16,
             std16)


def kernel(user_id, time_stamp, user_table, ts_table, boundaries, ts_mean,
           ts_std):
    mean16 = jnp.full((L,), ts_mean, dtype=jnp.float32)
    std16 = jnp.full((L,), ts_std, dtype=jnp.float32)
    return _run(user_id.astype(jnp.int32), time_stamp, user_table, ts_table,
                boundaries, mean16, std16)


# speedup vs baseline: 2.2284x; 1.3719x over previous
import jax
import jax.numpy as jnp
from jax import lax
from jax.experimental import pallas as pl
from jax.experimental.pallas import tpu as pltpu
from jax.experimental.pallas import tpu_sc as plsc

B = 16384
DIM = 64
NBND = 1000
TPAD = 1008
NC, NS, L = 2, 16, 16
NW = NC * NS
BPW = B // NW
NVEC = BPW // L
WAVE = 16
NWAVE = BPW // WAVE
NBUF = 2 * WAVE


def _fetch_wave(tab_hbm, ids_v, bufs, sem, w, p):
    iv = ids_v[pl.ds(pl.multiple_of(w * WAVE, WAVE), WAVE)]
    for e in range(WAVE):
        t0 = pl.multiple_of((iv[e] >> 3) * 8, 8)
        pltpu.async_copy(tab_hbm.at[pl.ds(t0, 8)],
                         bufs.at[p * WAVE + e], sem)


def _extract_wave(tab_hbm, ids_v, bufs, sem, emb_v, w, p):
    iv = ids_v[pl.ds(pl.multiple_of(w * WAVE, WAVE), WAVE)]
    for e in range(WAVE):
        pltpu.make_async_copy(tab_hbm.at[pl.ds(0, 8)],
                              bufs.at[p * WAVE + e], sem).wait()
        r = iv[e] & 7
        for q in range(DIM // L):
            emb_v[pl.ds((w * WAVE + e) * DIM + q * L, L)] = (
                bufs[p * WAVE + e, r, pl.ds(q * L, L)])


def _gather_phase(tab_hbm, ids_s, bufs, sem, emb_v):
    _fetch_wave(tab_hbm, ids_s, bufs, sem, 0, 0)
    _fetch_wave(tab_hbm, ids_s, bufs, sem, 1, 1)

    @pl.loop(0, NWAVE - 2, step=2)
    def _(w):
        for p in range(2):
            _extract_wave(tab_hbm, ids_s, bufs, sem, emb_v, w + p, p)
            _fetch_wave(tab_hbm, ids_s, bufs, sem, w + p + 2, p)
    for p in range(2):
        _extract_wave(tab_hbm, ids_s, bufs, sem, emb_v, NWAVE - 2 + p, p)


def _body(uid_hbm, ts_hbm, utab_hbm, ttab_hbm, bnd_hbm, mean_hbm, std_hbm,
          ue_hbm, te_hbm, nrm_hbm, idx_v, ts_v, bnd_v, bkt_v,
          nrm_v, mean_v, std_v, bufs, emb_v, gsem):
    wid = lax.axis_index("subcore") * NC + lax.axis_index("core")
    base = wid * BPW

    pltpu.sync_copy(uid_hbm.at[pl.ds(base, BPW)], idx_v)
    pltpu.sync_copy(ts_hbm.at[pl.ds(base, BPW)], ts_v)
    pltpu.sync_copy(bnd_hbm, bnd_v)
    pltpu.sync_copy(mean_hbm, mean_v)
    pltpu.sync_copy(std_hbm, std_v)
    mean = mean_v[...]
    std = std_v[...]

    @pl.loop(0, NVEC)
    def _(m):
        off = pl.multiple_of(m * L, L)
        ts = ts_v[pl.ds(off, L)]
        lo = jnp.zeros((L,), jnp.int32)
        hi = jnp.full((L,), NBND, jnp.int32)
        for _ in range(10):
            mid = (lo + hi) // 2
            bv = plsc.load_gather(bnd_v, [jnp.minimum(mid, NBND - 1)])
            act = lo < hi
            go = act & (bv <= ts)
            lo = jnp.where(go, mid + 1, lo)
            hi = jnp.where(act & jnp.logical_not(go), mid, hi)
        bkt_v[pl.ds(off, L)] = lo
        nrm_v[pl.ds(off, L)] = (ts - mean) / std

    _gather_phase(utab_hbm, idx_v, bufs, gsem, emb_v)
    pltpu.sync_copy(emb_v, ue_hbm.at[pl.ds(base * DIM, BPW * DIM)])

    _gather_phase(ttab_hbm, bkt_v, bufs, gsem, emb_v)
    pltpu.sync_copy(emb_v, te_hbm.at[pl.ds(base * DIM, BPW * DIM)])
    pltpu.sync_copy(nrm_v, nrm_hbm.at[pl.ds(base, BPW)])


@jax.jit
def _run(user_id, time_stamp, user_table, ts_table, boundaries, mean16,
         std16):
    mesh = plsc.VectorSubcoreMesh(core_axis_name="core",
                                  subcore_axis_name="subcore")
    f = pl.kernel(
        _body,
        out_type=(
            jax.ShapeDtypeStruct((B * DIM,), jnp.float32),
            jax.ShapeDtypeStruct((B * DIM,), jnp.float32),
            jax.ShapeDtypeStruct((B,), jnp.float32),
        ),
        mesh=mesh,
        scratch_types=[
            pltpu.VMEM((BPW,), jnp.int32),
            pltpu.VMEM((BPW,), jnp.float32),
            pltpu.VMEM((NBND,), jnp.float32),
            pltpu.VMEM((BPW,), jnp.int32),
            pltpu.VMEM((BPW,), jnp.float32),
            pltpu.VMEM((L,), jnp.float32),
            pltpu.VMEM((L,), jnp.float32),
            pltpu.VMEM((NBUF, 8, DIM), jnp.float32),
            pltpu.VMEM((BPW * DIM,), jnp.float32),
            pltpu.SemaphoreType.DMA,
        ],
        compiler_params=pltpu.CompilerParams(needs_layout_passes=False),
    )
    ue, te, nrm = f(user_id, time_stamp, user_table, ts_table, boundaries,
                    mean16, std16)
    return jnp.concatenate(
        [ue.reshape(B, DIM), te.reshape(B, DIM), nrm[:, None]], axis=1)


def kernel(user_id, time_stamp, user_table, ts_table, boundaries, ts_mean,
           ts_std):
    mean16 = jnp.full((L,), ts_mean, dtype=jnp.float32)
    std16 = jnp.full((L,), ts_std, dtype=jnp.float32)
    ttab = jnp.pad(ts_table, ((0, TPAD - ts_table.shape[0]), (0, 0)))
    return _run(user_id.astype(jnp.int32), time_stamp, user_table, ttab,
                boundaries, mean16, std16)


# speedup vs baseline: 4.0429x; 1.8143x over previous
import jax
import jax.numpy as jnp
from jax import lax
from jax.experimental import pallas as pl
from jax.experimental.pallas import tpu as pltpu
from jax.experimental.pallas import tpu_sc as plsc

B = 16384
DIM = 64
NBND = 1000
TTC = 1024
OUT_D = 2 * DIM + 1
NC, NS, L = 2, 16, 16
NW = NC * NS
BPW = B // NW
HALF = BPW // 2
NVEC = BPW // L
GRP = 16
NG = HALF // GRP
RING = 8


def _issue(tab_hbm, tc, bufs, slot, sem):
    col = pl.multiple_of(tc * 128, 128)
    pltpu.async_copy(tab_hbm.at[:, pl.ds(col, 128)], bufs.at[slot], sem)


def _wait_block(tab_hbm, bufs, slot, sem):
    pltpu.make_async_copy(tab_hbm.at[:, pl.ds(0, 128)], bufs.at[slot],
                          sem).wait()


def _extract(bufs, slot, lane, col, out_v, row0):
    lane_v = jnp.full((L,), lane, jnp.int32)
    slot_v = jnp.full((L,), slot, jnp.int32)
    col_v = jnp.full((L,), col, jnp.int32)
    for p in range(DIM // L):
        f_v = p * L + lax.iota(jnp.int32, L)
        vals = plsc.load_gather(bufs, [slot_v, f_v, lane_v])
        plsc.store_scatter(out_v, [row0 + f_v, col_v], vals)


def _body(uid_hbm, ts_hbm, utT_hbm, ttT_hbm, bnd_hbm, mean_hbm, std_hbm,
          outT_hbm, idx_v, ts_v, bnd_v, bkt_v, nrm_v, mean_v, std_v, bufs,
          out_v, gsem):
    wid = lax.axis_index("subcore") * NC + lax.axis_index("core")
    base = wid * BPW

    pltpu.sync_copy(uid_hbm.at[pl.ds(base, BPW)], idx_v)
    pltpu.sync_copy(ts_hbm.at[pl.ds(base, BPW)], ts_v)
    pltpu.sync_copy(bnd_hbm, bnd_v)
    pltpu.sync_copy(mean_hbm, mean_v)
    pltpu.sync_copy(std_hbm, std_v)
    mean = mean_v[...]
    std = std_v[...]

    @pl.loop(0, NVEC)
    def _(m):
        off = pl.multiple_of(m * L, L)
        ts = ts_v[pl.ds(off, L)]
        lo = jnp.zeros((L,), jnp.int32)
        hi = jnp.full((L,), NBND, jnp.int32)
        for _ in range(10):
            mid = (lo + hi) // 2
            bv = plsc.load_gather(bnd_v, [jnp.minimum(mid, NBND - 1)])
            act = lo < hi
            go = act & (bv <= ts)
            lo = jnp.where(go, mid + 1, lo)
            hi = jnp.where(act & jnp.logical_not(go), mid, hi)
        bkt_v[pl.ds(off, L)] = lo
        nrm_v[pl.ds(off, L)] = (ts - mean) / std

    for h in range(2):
        hoff = h * HALF

        iv0 = idx_v[pl.ds(pl.multiple_of(hoff, GRP), GRP)]
        for e in range(RING):
            _issue(utT_hbm, iv0[e] >> 7, bufs, e, gsem)

        @pl.loop(0, NG)
        def _(g):
            off = pl.multiple_of(hoff + g * GRP, GRP)
            iv = idx_v[pl.ds(off, GRP)]
            offn = pl.multiple_of(
                jnp.minimum(off + GRP, hoff + HALF - GRP), GRP)
            ivn = idx_v[pl.ds(offn, GRP)]
            for e in range(RING):
                _wait_block(utT_hbm, bufs, e, gsem)
                _extract(bufs, e, iv[e] & 127, g * GRP + e, out_v, 0)
                _issue(utT_hbm, iv[e + RING] >> 7, bufs, e, gsem)
            for e in range(RING):
                _wait_block(utT_hbm, bufs, e, gsem)
                _extract(bufs, e, iv[e + RING] & 127, g * GRP + e + RING,
                         out_v, 0)
                _issue(utT_hbm, ivn[e] >> 7, bufs, e, gsem)
        for e in range(RING):
            _wait_block(utT_hbm, bufs, e, gsem)

        for c in range(TTC // 128):
            pltpu.sync_copy(ttT_hbm.at[:, pl.ds(c * 128, 128)], bufs.at[c])

        @pl.loop(0, NG)
        def _(g):
            off = pl.multiple_of(hoff + g * GRP, GRP)
            bv = bkt_v[pl.ds(off, GRP)]
            for e in range(GRP):
                _extract(bufs, bv[e] >> 7, bv[e] & 127, g * GRP + e, out_v,
                         DIM)

        @pl.loop(0, HALF // L)
        def _(m):
            moff = pl.multiple_of(m * L, L)
            out_v[2 * DIM, pl.ds(moff, L)] = nrm_v[pl.ds(hoff + moff, L)]

        pltpu.sync_copy(out_v, outT_hbm.at[:, pl.ds(base + hoff, HALF)])


@jax.jit
def _run(user_id, time_stamp, utT, ttT, boundaries, mean16, std16):
    mesh = plsc.VectorSubcoreMesh(core_axis_name="core",
                                  subcore_axis_name="subcore")
    f = pl.kernel(
        _body,
        out_type=jax.ShapeDtypeStruct((OUT_D, B), jnp.float32),
        mesh=mesh,
        scratch_types=[
            pltpu.VMEM((BPW,), jnp.int32),
            pltpu.VMEM((BPW,), jnp.float32),
            pltpu.VMEM((NBND,), jnp.float32),
            pltpu.VMEM((BPW,), jnp.int32),
            pltpu.VMEM((BPW,), jnp.float32),
            pltpu.VMEM((L,), jnp.float32),
            pltpu.VMEM((L,), jnp.float32),
            pltpu.VMEM((RING, DIM, 128), jnp.float32),
            pltpu.VMEM((OUT_D, HALF), jnp.float32),
            pltpu.SemaphoreType.DMA,
        ],
        compiler_params=pltpu.CompilerParams(needs_layout_passes=False),
    )
    outT = f(user_id, time_stamp, utT, ttT, boundaries, mean16, std16)
    return outT.T


def kernel(user_id, time_stamp, user_table, ts_table, boundaries, ts_mean,
           ts_std):
    mean16 = jnp.full((L,), ts_mean, dtype=jnp.float32)
    std16 = jnp.full((L,), ts_std, dtype=jnp.float32)
    ttT = jnp.pad(ts_table, ((0, TTC - ts_table.shape[0]), (0, 0))).T
    return _run(user_id.astype(jnp.int32), time_stamp, user_table.T, ttT,
                boundaries, mean16, std16)
